# Initial kernel scaffold; baseline (speedup 1.0000x reference)
import jax
import jax.numpy as jnp
from jax.experimental import pallas as pl


def kernel(inputs, edge_index, W_self0, W_neigh0, b0, W_self1, W_neigh1, b1):
    raise NotImplementedError("write your pallas kernel here")


# speedup vs baseline: 6.9787x; 6.9787x over previous
import functools

import jax
import jax.numpy as jnp
from jax import lax
from jax.experimental import pallas as pl
from jax.experimental.pallas import tpu as pltpu
from jax.experimental.pallas import tpu_sc as plsc

N = 10000
E = 320000
D = 128
NPAD = 10240

NC = 2
NS = 16
NW = NC * NS
EW = E // NW
CH = 80
NCHUNK = EW // CH
RPS = NPAD // NS
ZCH = CH
NZ = RPS // ZCH

_MESH = dict(core_axis_name="c", subcore_axis_name="s",
             num_cores=NC, num_subcores=NS)


def _msg_body(h, src3, dst3, zrow_h, P,
              acc, src_v, dst_v, rows_v, sem):
    cid = lax.axis_index("c")
    sid = lax.axis_index("s")
    wid = cid * NS + sid

    pltpu.sync_copy(src3.at[wid], src_v)
    pltpu.sync_copy(dst3.at[wid], dst_v)

    pltpu.sync_copy(zrow_h, rows_v)
    for i in range(NZ):
        pltpu.sync_copy(rows_v, acc.at[pl.ds(sid * RPS + i * ZCH, ZCH)])

    plsc.subcore_barrier()

    def chunk(j, _):
        pltpu.async_copy(h.at[src_v.at[j]], rows_v, sem).wait()
        pltpu.sync_copy(rows_v, acc.at[dst_v.at[j]], add=True)
        return ()

    lax.fori_loop(0, NCHUNK, chunk, (), unroll=False)

    plsc.subcore_barrier()

    for i in range(NZ):
        r0 = sid * RPS + i * ZCH
        pltpu.sync_copy(acc.at[pl.ds(r0, ZCH)], rows_v)
        pltpu.sync_copy(rows_v, P.at[cid, pl.ds(r0, ZCH)])


@functools.lru_cache(maxsize=None)
def _get_sc_msg():
  return pl.kernel(
    _msg_body,
    out_type=jax.ShapeDtypeStruct((NC, NPAD, D), jnp.float32),
    mesh=plsc.VectorSubcoreMesh(**_MESH),
    scratch_types=[
        pltpu.VMEM_SHARED((NPAD, D), jnp.float32),
        pltpu.VMEM((NCHUNK, CH), jnp.int32),
        pltpu.VMEM((NCHUNK, CH), jnp.int32),
        pltpu.VMEM((CH, D), jnp.float32),
        pltpu.SemaphoreType.DMA,
    ],
  )


def _deg_body(dst3, zrow_h, DEG, dst_v, hist_v):
    cid = lax.axis_index("c")
    sid = lax.axis_index("s")
    wid = cid * NS + sid

    pltpu.sync_copy(dst3.at[wid], dst_v)
    pltpu.sync_copy(zrow_h, hist_v)

    ones = jnp.ones((16,), jnp.float32)

    def chunk(j, _):
        for k in range(CH // 16):
            dv = dst_v[j, pl.ds(k * 16, 16)]
            hi = lax.shift_right_logical(dv, 7)
            lo = lax.bitwise_and(dv, 127)
            plsc.addupdate_scatter(hist_v, [hi, lo], ones)
        return ()

    lax.fori_loop(0, NCHUNK, chunk, (), unroll=False)

    pltpu.sync_copy(hist_v, DEG.at[wid])


@functools.lru_cache(maxsize=None)
def _get_sc_deg():
  return pl.kernel(
    _deg_body,
    out_type=jax.ShapeDtypeStruct((NW, NPAD // D, D), jnp.float32),
    mesh=plsc.VectorSubcoreMesh(**_MESH),
    scratch_types=[
        pltpu.VMEM((NCHUNK, CH), jnp.int32),
        pltpu.VMEM((NPAD // D, D), jnp.float32),
    ],
    compiler_params=pltpu.CompilerParams(needs_layout_passes=False),
  )


def _tc_body(relu, x_ref, p_ref, deg_ref, ws_ref, wn_ref, b_ref, o_ref):
    s = p_ref[0] + p_ref[1]
    d = jnp.sum(deg_ref[...], axis=0)
    hn = s / jnp.maximum(d, 1.0)[:, None]
    o = (jnp.dot(x_ref[...], ws_ref[...], preferred_element_type=jnp.float32)
         + jnp.dot(hn, wn_ref[...], preferred_element_type=jnp.float32)
         + b_ref[...])
    if relu:
        o = jnp.maximum(o, 0.0)
    o_ref[...] = o


_TCB = 512


def _tc_dense(x, P, DEG, W_self, W_neigh, b, relu):
    grid = (NPAD // _TCB,)
    return pl.pallas_call(
        functools.partial(_tc_body, relu),
        grid=grid,
        in_specs=[
            pl.BlockSpec((_TCB, D), lambda i: (i, 0)),
            pl.BlockSpec((NC, _TCB, D), lambda i: (0, i, 0)),
            pl.BlockSpec((NW, _TCB), lambda i: (0, i)),
            pl.BlockSpec((D, D), lambda i: (0, 0)),
            pl.BlockSpec((D, D), lambda i: (0, 0)),
            pl.BlockSpec((1, D), lambda i: (0, 0)),
        ],
        out_specs=pl.BlockSpec((_TCB, D), lambda i: (i, 0)),
        out_shape=jax.ShapeDtypeStruct((NPAD, D), jnp.float32),
    )(x, P, DEG, W_self, W_neigh, b.reshape(1, D))


def kernel(inputs, edge_index, W_self0, W_neigh0, b0, W_self1, W_neigh1, b1):
    src3 = edge_index[0].reshape(NW, NCHUNK, CH)
    dst3 = edge_index[1].reshape(NW, NCHUNK, CH)
    zrow = jnp.zeros((ZCH, D), jnp.float32)

    xp = jnp.zeros((NPAD, D), jnp.float32).at[:N].set(inputs)
    DEG = _get_sc_deg()(dst3, zrow).reshape(NW, NPAD)
    P0 = _get_sc_msg()(xp, src3, dst3, zrow)
    h1 = _tc_dense(xp, P0, DEG, W_self0, W_neigh0, b0, relu=True)
    P1 = _get_sc_msg()(h1, src3, dst3, zrow)
    out = _tc_dense(h1, P1, DEG, W_self1, W_neigh1, b1, relu=False)
    return out[:N]
